# baseline (speedup 1.0000x reference)
import functools

import jax
import jax.numpy as jnp
from jax import lax
from jax.experimental import pallas as pl
from jax.experimental.pallas import tpu as pltpu
from jax.experimental.pallas import tpu_sc as plsc

DIM = 64
NC = 2
NS = 16
NW = NC * NS
BLK = 128
DUNROLL = 8


@functools.partial(jax.jit, static_argnames=("fields", "batch"))
def _gather(tbl128, idx, fields, batch):
    nb = batch // BLK
    blk_per_w = fields * nb // NW
    niter = blk_per_w // 2
    mesh = plsc.VectorSubcoreMesh(core_axis_name="c", subcore_axis_name="s")

    @functools.partial(
        pl.kernel,
        mesh=mesh,
        out_type=jax.ShapeDtypeStruct((fields, DIM, batch), jnp.float32),
        compiler_params=pltpu.CompilerParams(needs_layout_passes=False),
        scratch_types=[
            pltpu.VMEM((BLK,), jnp.int32),
            pltpu.VMEM((BLK,), jnp.int32),
            pltpu.VMEM((BLK, 129), jnp.float32),
            pltpu.VMEM((BLK, 129), jnp.float32),
            pltpu.VMEM((DIM, BLK), jnp.float32),
            pltpu.VMEM((DIM, BLK), jnp.float32),
            pltpu.SemaphoreType.DMA,
            pltpu.SemaphoreType.DMA,
            pltpu.SemaphoreType.DMA,
            pltpu.SemaphoreType.DMA,
        ],
    )
    def gather_k(tbl_hbm, idx_hbm, out_hbm,
                 idx_v0, idx_v1, rows_v0, rows_v1, trans_v0, trans_v1,
                 gsem0, gsem1, osem0, osem1):
        wid = lax.axis_index("s") * NC + lax.axis_index("c")
        g0 = wid * blk_per_w
        slots = ((idx_v0, rows_v0, trans_v0, gsem0, osem0),
                 (idx_v1, rows_v1, trans_v1, gsem1, osem1))

        def fetch(g, idx_v, rows_v, gsem):
            off = (g // nb) * batch + (g % nb) * BLK
            pltpu.sync_copy(idx_hbm.at[pl.ds(off, BLK)], idx_v)
            pltpu.make_async_copy(tbl_hbm.at[idx_v], rows_v.at[:, pl.ds(0, 128)], gsem).start()

        for s in range(2):
            idx_v, rows_v, _, gsem, _ = slots[s]
            fetch(g0 + s, idx_v, rows_v, gsem)

        def body(i, carry):
            for s in range(2):
                idx_v, rows_v, trans_v, gsem, osem = slots[s]
                g = g0 + 2 * i + s
                f = g // nb
                bb = g % nb
                pltpu.make_async_copy(tbl_hbm.at[idx_v], rows_v.at[:, pl.ds(0, 128)], gsem).wait()

                @pl.when(i > 0)
                def _drain():
                    pltpu.make_async_copy(
                        out_hbm.at[0, :, pl.ds(0, BLK)], trans_v, osem).wait()

                def dloop(d0, c2):
                    d = d0 * DUNROLL
                    for dd in range(DUNROLL):
                        dvec = jnp.full((16,), dd, jnp.int32) + d
                        vals = []
                        for j in range(BLK // 16):
                            ridx = lax.iota(jnp.int32, 16) + (16 * j)
                            vals.append(plsc.load_gather(rows_v, [ridx, dvec]))
                        for j in range(BLK // 16):
                            trans_v[d + dd, pl.ds(16 * j, 16)] = vals[j]
                    return c2

                lax.fori_loop(0, DIM // DUNROLL, dloop, 0)
                pltpu.make_async_copy(
                    trans_v, out_hbm.at[f, :, pl.ds(bb * BLK, BLK)], osem
                ).start()

                @pl.when(i < niter - 1)
                def _prefetch():
                    fetch(g + 2, idx_v, rows_v, gsem)
            return carry

        lax.fori_loop(0, niter, body, 0)
        for s in range(2):
            _, _, trans_v, _, osem = slots[s]
            pltpu.make_async_copy(
                out_hbm.at[0, :, pl.ds(0, BLK)], trans_v, osem).wait()

    return gather_k(tbl128, idx)


def kernel(cat_ids, table):
    batch, fields = cat_ids.shape
    idx = cat_ids.T.reshape(batch * fields).astype(jnp.int32)
    tbl128 = jnp.pad(table, ((0, 0), (0, 128 - DIM)))
    out3 = _gather(tbl128, idx, fields, batch)
    return out3.transpose(2, 0, 1)


# speedup vs baseline: 1.0410x; 1.0410x over previous
import functools

import jax
import jax.numpy as jnp
from jax import lax
from jax.experimental import pallas as pl
from jax.experimental.pallas import tpu as pltpu
from jax.experimental.pallas import tpu_sc as plsc

DIM = 64
NC = 2
NS = 16
NW = NC * NS
BLK = 256
DUNROLL = 8


@functools.partial(jax.jit, static_argnames=("fields", "batch"))
def _gather(tbl128, idx, fields, batch):
    nb = batch // BLK
    blk_per_w = fields * nb // NW
    idx_per_w = fields * batch // NW
    niter = blk_per_w // 2
    mesh = plsc.VectorSubcoreMesh(core_axis_name="c", subcore_axis_name="s")

    @functools.partial(
        pl.kernel,
        mesh=mesh,
        out_type=jax.ShapeDtypeStruct((fields, DIM, batch), jnp.float32),
        compiler_params=pltpu.CompilerParams(needs_layout_passes=False),
        scratch_types=[
            pltpu.VMEM((fields * batch // NW,), jnp.int32),
            pltpu.VMEM((BLK, 128), jnp.float32),
            pltpu.VMEM((BLK, 128), jnp.float32),
            pltpu.VMEM((DIM, BLK), jnp.float32),
            pltpu.VMEM((DIM, BLK), jnp.float32),
            pltpu.SemaphoreType.DMA,
            pltpu.SemaphoreType.DMA,
            pltpu.SemaphoreType.DMA,
            pltpu.SemaphoreType.DMA,
        ],
    )
    def gather_k(tbl_hbm, idx_hbm, out_hbm,
                 idx_all, rows_v0, rows_v1, trans_v0, trans_v1,
                 gsem0, gsem1, osem0, osem1):
        wid = lax.axis_index("s") * NC + lax.axis_index("c")
        g0 = wid * blk_per_w
        slots = ((rows_v0, trans_v0, gsem0, osem0),
                 (rows_v1, trans_v1, gsem1, osem1))

        pltpu.sync_copy(idx_hbm.at[pl.ds(wid * idx_per_w, idx_per_w)], idx_all)

        def start_gather(k, rows_v, gsem):
            pltpu.make_async_copy(
                tbl_hbm.at[idx_all.at[pl.ds(k * BLK, BLK)]], rows_v, gsem
            ).start()

        def wait_gather(k, rows_v, gsem):
            pltpu.make_async_copy(
                tbl_hbm.at[idx_all.at[pl.ds(k * BLK, BLK)]], rows_v, gsem
            ).wait()

        for s in range(2):
            rows_v, _, gsem, _ = slots[s]
            start_gather(s, rows_v, gsem)

        def body(i, carry):
            for s in range(2):
                rows_v, trans_v, gsem, osem = slots[s]
                k = 2 * i + s
                g = g0 + k
                f = g // nb
                bb = g % nb
                wait_gather(k, rows_v, gsem)

                @pl.when(i > 0)
                def _drain():
                    pltpu.make_async_copy(
                        out_hbm.at[0, :, pl.ds(0, BLK)], trans_v, osem).wait()

                def dloop(d0, c2):
                    d = d0 * DUNROLL
                    for dd in range(DUNROLL):
                        dvec = jnp.full((16,), dd, jnp.int32) + d
                        vals = []
                        for j in range(BLK // 16):
                            ridx = lax.iota(jnp.int32, 16) + (16 * j)
                            vals.append(plsc.load_gather(rows_v, [ridx, dvec]))
                        for j in range(BLK // 16):
                            trans_v[d + dd, pl.ds(16 * j, 16)] = vals[j]
                    return c2

                lax.fori_loop(0, DIM // DUNROLL, dloop, 0)
                pltpu.make_async_copy(
                    trans_v, out_hbm.at[f, :, pl.ds(bb * BLK, BLK)], osem
                ).start()

                @pl.when(i < niter - 1)
                def _prefetch():
                    start_gather(k + 2, rows_v, gsem)
            return carry

        lax.fori_loop(0, niter, body, 0)
        for s in range(2):
            _, trans_v, _, osem = slots[s]
            pltpu.make_async_copy(
                out_hbm.at[0, :, pl.ds(0, BLK)], trans_v, osem).wait()

    return gather_k(tbl128, idx)


def kernel(cat_ids, table):
    batch, fields = cat_ids.shape
    idx = cat_ids.T.reshape(batch * fields).astype(jnp.int32)
    tbl128 = jnp.pad(table, ((0, 0), (0, 128 - DIM)))
    out3 = _gather(tbl128, idx, fields, batch)
    return out3.transpose(2, 0, 1)
